# baseline (speedup 1.0000x reference)
import functools

import jax
import jax.numpy as jnp
from jax.experimental import pallas as pl
from jax.experimental.pallas import tpu as pltpu

_HID = 32
_IN = 4
_ACT = 2
_LANES = 128
_GROUPS = 4
_BX = 512


def _prep(slab):
    w1 = slab[0:_IN, 0:_HID]
    w2 = slab[16:16 + _HID, 0:_HID]
    w3 = slab[48:48 + _HID, 0:_ACT]
    b1 = slab[8, 0:_HID]
    b2 = slab[9, 0:_HID]
    b3 = slab[10, 0:_ACT]

    wa = jnp.kron(jnp.eye(32, dtype=jnp.float32), w1)
    was = jnp.concatenate(
        [wa[:, 256 * p:256 * (p + 1)] for p in range(_GROUPS)], axis=0)
    w2p = jnp.kron(jnp.eye(8, dtype=jnp.float32), w2)
    wc_base = jnp.kron(jnp.eye(8, dtype=jnp.float32), w3)
    wcs = jnp.concatenate(
        [jnp.pad(wc_base, ((0, 0), (16 * p, 112 - 16 * p)))
         for p in range(_GROUPS)], axis=0)

    bias = jnp.zeros((8, 256), jnp.float32)
    bias = bias.at[0, :].set(jnp.tile(b1, 8))
    bias = bias.at[1, :].set(jnp.tile(b2, 8))
    bias = bias.at[2, :128].set(jnp.tile(b3, 64))
    return (was.astype(jnp.bfloat16), w2p.astype(jnp.bfloat16),
            wcs.astype(jnp.bfloat16), bias)


def _mlp_body(x_ref, was_ref, w2_ref, wcs_ref, b_ref, q_ref, p_scr):
    xv = x_ref[...].astype(jnp.bfloat16)
    b1 = b_ref[0:1, :]
    b2 = b_ref[1:2, :]
    b3 = b_ref[2:3, 0:_LANES]
    w2 = w2_ref[...]

    acc = jnp.zeros((_BX, _LANES), jnp.float32)
    for p in range(_GROUPS):
        wa = was_ref[128 * p:128 * (p + 1), :]
        wc = wcs_ref[256 * p:256 * (p + 1), :]
        h1 = jnp.maximum(
            jnp.dot(xv, wa, preferred_element_type=jnp.float32) + b1, 0.0)
        h2 = jnp.maximum(
            jnp.dot(h1.astype(jnp.bfloat16), w2,
                    preferred_element_type=jnp.float32) + b2, 0.0)
        acc = acc + jnp.dot(h2.astype(jnp.bfloat16), wc,
                            preferred_element_type=jnp.float32)
    p_scr[...] = acc
    ev = p_scr[pl.ds(0, _BX // 2, stride=2), :]
    od = p_scr[pl.ds(1, _BX // 2, stride=2), :]
    q_ref[...] = ev + jnp.concatenate(
        [od[:, 64:], od[:, :64]], axis=1) + b3


def kernel(x, slab):
    batch = x.shape[0]
    xd = x.reshape(batch // 32, _LANES)
    rows = xd.shape[0]
    was, w2p, wcs, bias = _prep(slab)
    grid = rows // _BX

    flops = 2 * batch * (_IN * _HID + _HID * _HID + _HID * _ACT)
    cost = pl.CostEstimate(flops=flops, transcendentals=0,
                           bytes_accessed=x.size * 4 + batch * _ACT * 4)

    qd = pl.pallas_call(
        _mlp_body,
        out_shape=jax.ShapeDtypeStruct((rows // 2, _LANES), jnp.float32),
        grid=(grid,),
        in_specs=[pl.BlockSpec((_BX, _LANES), lambda i: (i, 0)),
                  pl.BlockSpec((512, 256), lambda i: (0, 0)),
                  pl.BlockSpec((256, 256), lambda i: (0, 0)),
                  pl.BlockSpec((1024, _LANES), lambda i: (0, 0)),
                  pl.BlockSpec((8, 256), lambda i: (0, 0))],
        out_specs=pl.BlockSpec((_BX // 2, _LANES), lambda i: (i, 0)),
        scratch_shapes=[pltpu.VMEM((_BX, _LANES), jnp.float32)],
        compiler_params=pltpu.CompilerParams(
            dimension_semantics=("parallel",)),
        cost_estimate=cost,
    )(xd, was, w2p, wcs, bias)
    return qd.reshape(batch, _ACT)


# speedup vs baseline: 6.6862x; 6.6862x over previous
import jax
import jax.numpy as jnp
from jax.experimental import pallas as pl
from jax.experimental.pallas import tpu as pltpu

_HID = 32
_IN = 4
_ACT = 2
_LB = 2048


def _mlp_body(x_ref, w1_ref, w2_ref, w3_ref, b_ref, q_ref):
    xv = x_ref[...].astype(jnp.bfloat16)
    b1 = b_ref[:, 0:1]
    b2 = b_ref[:, 1:2]
    b3 = b_ref[0:_ACT, 2:3]

    h1 = jnp.maximum(
        jnp.dot(w1_ref[...], xv, preferred_element_type=jnp.float32) + b1,
        0.0).astype(jnp.bfloat16)
    h2 = jnp.maximum(
        jnp.dot(w2_ref[...], h1, preferred_element_type=jnp.float32) + b2,
        0.0).astype(jnp.bfloat16)
    q_ref[...] = jnp.dot(
        w3_ref[...], h2, preferred_element_type=jnp.float32) + b3


def kernel(x, slab):
    batch = x.shape[0]
    xt = jnp.transpose(x)
    w1t = jnp.transpose(slab[0:_IN, 0:_HID]).astype(jnp.bfloat16)
    w2t = jnp.transpose(slab[16:16 + _HID, 0:_HID]).astype(jnp.bfloat16)
    w3t = jnp.transpose(slab[48:48 + _HID, 0:_ACT]).astype(jnp.bfloat16)
    bias = jnp.zeros((_HID, 8), jnp.float32)
    bias = bias.at[:, 0].set(slab[8, 0:_HID])
    bias = bias.at[:, 1].set(slab[9, 0:_HID])
    bias = bias.at[0:_ACT, 2].set(slab[10, 0:_ACT])

    grid = batch // _LB
    flops = 2 * batch * (_IN * _HID + _HID * _HID + _HID * _ACT)
    cost = pl.CostEstimate(flops=flops, transcendentals=0,
                           bytes_accessed=x.size * 4 + batch * _ACT * 4)

    qt = pl.pallas_call(
        _mlp_body,
        out_shape=jax.ShapeDtypeStruct((_ACT, batch), jnp.float32),
        grid=(grid,),
        in_specs=[pl.BlockSpec((_IN, _LB), lambda i: (0, i)),
                  pl.BlockSpec((_HID, _IN), lambda i: (0, 0)),
                  pl.BlockSpec((_HID, _HID), lambda i: (0, 0)),
                  pl.BlockSpec((_ACT, _HID), lambda i: (0, 0)),
                  pl.BlockSpec((_HID, 8), lambda i: (0, 0))],
        out_specs=pl.BlockSpec((_ACT, _LB), lambda i: (0, i)),
        compiler_params=pltpu.CompilerParams(
            dimension_semantics=("parallel",)),
        cost_estimate=cost,
    )(xt, w1t, w2t, w3t, bias)
    return jnp.transpose(qt)


# speedup vs baseline: 39.1731x; 5.8588x over previous
import jax
import jax.numpy as jnp
from jax.experimental import pallas as pl
from jax.experimental.pallas import tpu as pltpu

_HID = 32
_IN = 4
_ACT = 2
_LB = 32768


def _mlp_body(x_ref, w1_ref, w2_ref, w3_ref, b_ref, q_ref):
    xv = x_ref[...].astype(jnp.bfloat16)
    b1 = b_ref[:, 0:1]
    b2 = b_ref[:, 1:2]
    b3 = b_ref[0:_ACT, 2:3]

    h1 = jnp.maximum(
        jnp.dot(w1_ref[...], xv, preferred_element_type=jnp.float32) + b1,
        0.0).astype(jnp.bfloat16)
    h2 = jnp.maximum(
        jnp.dot(w2_ref[...], h1, preferred_element_type=jnp.float32) + b2,
        0.0).astype(jnp.bfloat16)
    q_ref[...] = jnp.dot(
        w3_ref[...], h2, preferred_element_type=jnp.float32) + b3


def kernel(x, slab):
    batch = x.shape[0]
    xt = jnp.transpose(x)
    w1t = jnp.transpose(slab[0:_IN, 0:_HID]).astype(jnp.bfloat16)
    w2t = jnp.transpose(slab[16:16 + _HID, 0:_HID]).astype(jnp.bfloat16)
    w3t = jnp.transpose(slab[48:48 + _HID, 0:_ACT]).astype(jnp.bfloat16)
    bias = jnp.zeros((_HID, 8), jnp.float32)
    bias = bias.at[:, 0].set(slab[8, 0:_HID])
    bias = bias.at[:, 1].set(slab[9, 0:_HID])
    bias = bias.at[0:_ACT, 2].set(slab[10, 0:_ACT])

    grid = batch // _LB
    flops = 2 * batch * (_IN * _HID + _HID * _HID + _HID * _ACT)
    cost = pl.CostEstimate(flops=flops, transcendentals=0,
                           bytes_accessed=x.size * 4 + batch * _ACT * 4)

    qt = pl.pallas_call(
        _mlp_body,
        out_shape=jax.ShapeDtypeStruct((_ACT, batch), jnp.float32),
        grid=(grid,),
        in_specs=[pl.BlockSpec((_IN, _LB), lambda i: (0, i)),
                  pl.BlockSpec((_HID, _IN), lambda i: (0, 0)),
                  pl.BlockSpec((_HID, _HID), lambda i: (0, 0)),
                  pl.BlockSpec((_ACT, _HID), lambda i: (0, 0)),
                  pl.BlockSpec((_HID, 8), lambda i: (0, 0))],
        out_specs=pl.BlockSpec((_ACT, _LB), lambda i: (0, i)),
        compiler_params=pltpu.CompilerParams(
            dimension_semantics=("parallel",)),
        cost_estimate=cost,
    )(xt, w1t, w2t, w3t, bias)
    return jnp.transpose(qt)


# speedup vs baseline: 46.3787x; 1.1839x over previous
import jax
import jax.numpy as jnp
from jax.experimental import pallas as pl
from jax.experimental.pallas import tpu as pltpu

_HID = 32
_IN = 4
_ACT = 2
_LB = 262144

_OFF_W1 = 0
_OFF_B = 8
_OFF_W2 = 16
_OFF_W3 = 48
_SLAB_ROWS = 80

def _mlp_body(x_ref, s_ref, q_ref):
    xv = x_ref[...].astype(jnp.bfloat16)
    w1t = s_ref[0:_HID, _OFF_W1:_OFF_W1 + _IN].astype(jnp.bfloat16)
    w2t = s_ref[0:_HID, _OFF_W2:_OFF_W2 + _HID].astype(jnp.bfloat16)
    w3t = s_ref[0:_ACT, _OFF_W3:_OFF_W3 + _HID].astype(jnp.bfloat16)
    b1 = s_ref[0:_HID, _OFF_B + 0:_OFF_B + 1].astype(jnp.bfloat16)
    b2 = s_ref[0:_HID, _OFF_B + 1:_OFF_B + 2].astype(jnp.bfloat16)
    b3 = s_ref[0:_ACT, _OFF_B + 2:_OFF_B + 3]

    h1 = jnp.maximum(
        jnp.dot(w1t, xv, preferred_element_type=jnp.float32)
        .astype(jnp.bfloat16) + b1, 0)
    h2 = jnp.maximum(
        jnp.dot(w2t, h1, preferred_element_type=jnp.float32)
        .astype(jnp.bfloat16) + b2, 0)
    q_ref[...] = jnp.dot(
        w3t, h2, preferred_element_type=jnp.float32) + b3


def kernel(x, slab):
    batch = x.shape[0]
    xt = jnp.transpose(x)
    st = jnp.transpose(slab)

    lb = _LB
    while batch % lb:
        lb //= 2
    grid = batch // lb
    flops = 2 * batch * (_IN * _HID + _HID * _HID + _HID * _ACT)
    cost = pl.CostEstimate(flops=flops, transcendentals=0,
                           bytes_accessed=x.size * 4 + batch * _ACT * 4)

    qt = pl.pallas_call(
        _mlp_body,
        out_shape=jax.ShapeDtypeStruct((_ACT, batch), jnp.float32),
        grid=(grid,),
        in_specs=[pl.BlockSpec((_IN, lb), lambda i: (0, i)),
                  pl.BlockSpec((128, _SLAB_ROWS), lambda i: (0, 0))],
        out_specs=pl.BlockSpec((_ACT, lb), lambda i: (0, i)),
        compiler_params=pltpu.CompilerParams(
            dimension_semantics=("arbitrary",)),
        cost_estimate=cost,
    )(xt, st)
    return jnp.transpose(qt)
